# baseline (speedup 1.0000x reference)
import functools

import jax
import jax.numpy as jnp
from jax import lax
from jax.experimental import pallas as pl
from jax.experimental.pallas import tpu as pltpu
from jax.experimental.pallas import tpu_sc as plsc

B = 128
CH, H, W = 3, 224, 224
NC, NS = 2, 16
NW = NC * NS
SPW = B // NW
RB = 112
NRB = H // RB
VPR = W // 16

_mesh = plsc.VectorSubcoreMesh(core_axis_name="c", subcore_axis_name="s")


@functools.partial(
    pl.kernel,
    mesh=_mesh,
    compiler_params=pltpu.CompilerParams(use_tc_tiling_on_sc=True,
                                         needs_layout_passes=True),
    out_type=jax.ShapeDtypeStruct((B, CH, H, W), jnp.float32),
    scratch_types=[
        pltpu.VMEM((B * 16,), jnp.float32),
        pltpu.VMEM((RB, W), jnp.float32),
        pltpu.VMEM((RB, W), jnp.float32),
        pltpu.VMEM((RB, W), jnp.float32),
        pltpu.VMEM((RB, W), jnp.float32),
        pltpu.SemaphoreType.DMA,
        pltpu.SemaphoreType.DMA,
        pltpu.SemaphoreType.DMA,
        pltpu.SemaphoreType.DMA,
    ],
)
def _scale_kernel(x_hbm, meff_hbm, out_hbm, meff_v, ib0, ib1, ob0, ob1,
                  si0, si1, so0, so1):
    w = lax.axis_index("s") * NC + lax.axis_index("c")
    pltpu.sync_copy(meff_hbm, meff_v)

    ibs, obs = [ib0, ib1], [ob0, ob1]
    sis, sos = [si0, si1], [so0, so1]
    h_in, h_out = [None, None], [None, None]

    chunks = [(s, c, r) for s in range(SPW) for c in range(CH)
              for r in range(NRB)]
    N = len(chunks)

    def src(k):
        s, c, r = chunks[k]
        return x_hbm.at[w * SPW + s, c, pl.ds(r * RB, RB), :]

    def dst(k):
        s, c, r = chunks[k]
        return out_hbm.at[w * SPW + s, c, pl.ds(r * RB, RB), :]

    h_in[0] = pltpu.async_copy(src(0), ibs[0], sis[0])
    for k in range(N):
        b = k % 2
        if k + 1 < N:
            h_in[1 - b] = pltpu.async_copy(src(k + 1), ibs[1 - b],
                                           sis[1 - b])
        if h_out[b] is not None:
            h_out[b].wait()
        h_in[b].wait()

        sid = w * SPW + chunks[k][0]
        m = meff_v[pl.ds(sid * 16, 16)]
        ib, ob = ibs[b], obs[b]

        @plsc.parallel_loop(0, RB, 1, unroll=2)
        def body(r, ib=ib, ob=ob, m=m):
            for u in range(VPR):
                sl = pl.ds(u * 16, 16)
                ob[r, sl] = ib[r, sl] * m

        h_out[b] = pltpu.async_copy(obs[b], dst(k), sos[b])

    h_out[0].wait()
    h_out[1].wait()


def kernel(input, magnitudes, probs):
    m_eff = jnp.where(probs, magnitudes, jnp.float32(1.0))
    m_splat = jnp.broadcast_to(m_eff[:, None], (B, 16)).reshape(B * 16)
    return _scale_kernel(input, m_splat)


# speedup vs baseline: 2.9646x; 2.9646x over previous
import functools

import jax
import jax.numpy as jnp
from jax import lax
from jax.experimental import pallas as pl
from jax.experimental.pallas import tpu as pltpu
from jax.experimental.pallas import tpu_sc as plsc

B = 128
CH, H, W = 3, 224, 224
NROW = CH * H
RW = W * B
NC, NS = 2, 16
NW = NC * NS
RPW = NROW // NW

_mesh = plsc.VectorSubcoreMesh(core_axis_name="c", subcore_axis_name="s")


@functools.partial(
    pl.kernel,
    mesh=_mesh,
    compiler_params=pltpu.CompilerParams(use_tc_tiling_on_sc=False),
    out_type=jax.ShapeDtypeStruct((NROW, RW), jnp.float32),
    scratch_types=[
        pltpu.VMEM((B,), jnp.float32),
        pltpu.VMEM((RW,), jnp.float32),
        pltpu.VMEM((RW,), jnp.float32),
        pltpu.VMEM((RW,), jnp.float32),
        pltpu.VMEM((RW,), jnp.float32),
        pltpu.SemaphoreType.DMA,
        pltpu.SemaphoreType.DMA,
        pltpu.SemaphoreType.DMA,
        pltpu.SemaphoreType.DMA,
    ],
)
def _scale_kernel(x_hbm, meff_hbm, out_hbm, meff_v, ib0, ib1, ob0, ob1,
                  si0, si1, so0, so1):
    w = lax.axis_index("s") * NC + lax.axis_index("c")
    base = w * RPW
    pltpu.sync_copy(meff_hbm, meff_v)
    mvec = [meff_v[pl.ds(u * 16, 16)] for u in range(8)]

    ibs, obs = [ib0, ib1], [ob0, ob1]
    sis, sos = [si0, si1], [so0, so1]
    h_in, h_out = [None, None], [None, None]

    h_in[0] = pltpu.async_copy(x_hbm.at[base], ibs[0], sis[0])
    for k in range(RPW):
        b = k % 2
        if k + 1 < RPW:
            h_in[1 - b] = pltpu.async_copy(x_hbm.at[base + k + 1],
                                           ibs[1 - b], sis[1 - b])
        if h_out[b] is not None:
            h_out[b].wait()
        h_in[b].wait()

        ib, ob = ibs[b], obs[b]

        @plsc.parallel_loop(0, RW, B, unroll=2)
        def body(i, ib=ib, ob=ob):
            for u in range(8):
                sl = pl.ds(i + u * 16, 16)
                ob[sl] = ib[sl] * mvec[u]

        h_out[b] = pltpu.async_copy(obs[b], out_hbm.at[base + k], sos[b])

    h_out[0].wait()
    h_out[1].wait()


def kernel(input, magnitudes, probs):
    m_eff = jnp.where(probs, magnitudes, jnp.float32(1.0))
    x_t = jnp.transpose(input, (1, 2, 3, 0)).reshape(NROW, RW)
    out = _scale_kernel(x_t, m_eff)
    return jnp.transpose(out.reshape(CH, H, W, B), (3, 0, 1, 2))
